# baseline (speedup 1.0000x reference)
import jax
import jax.numpy as jnp
from jax.experimental import pallas as pl

_DIM = 384
_N = 1024
_K = 10
_PIX = 1024


def _block_body(x_ref, mem_ref, out_ref):
    x = x_ref[0]
    mem = mem_ref[...]
    l = jax.lax.dot_general(mem, x, (((1,), (0,)), ((), ())),
                            preferred_element_type=jnp.float32)
    half = _N // 2
    c1 = jnp.maximum(l[:half], l[half:])
    c2 = jnp.minimum(l[:half], l[half:])
    m = None
    for i in range(_K):
        t = jnp.max(c1, axis=0, keepdims=True)
        if i == 0:
            m = t
        hit = c1 >= t
        c1 = jnp.where(hit, c2, c1)
        c2 = jnp.where(hit, -jnp.inf, c2)
    e = jnp.exp(l - m)
    z = jnp.sum(e, axis=0, keepdims=True)
    p = e * (1.0 / z)
    w = jnp.where(l >= t, jnp.exp(p), 0.0)
    w = w * (1.0 / jnp.sum(w, axis=0, keepdims=True))
    out_ref[0] = jax.lax.dot_general(mem, w, (((0,), (0,)), ((), ())),
                                     preferred_element_type=jnp.float32)


def _branch(xf, mempool, interpret=False):
    b = xf.shape[0]
    return pl.pallas_call(
        _block_body,
        grid=(b,),
        in_specs=[
            pl.BlockSpec((1, _DIM, _PIX), lambda i: (i, 0, 0)),
            pl.BlockSpec((_N, _DIM), lambda i: (0, 0)),
        ],
        out_specs=pl.BlockSpec((1, _DIM, _PIX), lambda i: (i, 0, 0)),
        out_shape=jax.ShapeDtypeStruct((b, _DIM, _PIX), jnp.float32),
        interpret=interpret,
    )(xf, mempool)


def kernel(input1, input2, mempool):
    outs = []
    for x in (input1, input2):
        b, c, h, w = x.shape
        o = _branch(x.reshape(b, c, h * w), mempool)
        outs.append(o.reshape(b, c, h, w))
    return tuple(outs)


# speedup vs baseline: 1.1671x; 1.1671x over previous
import jax
import jax.numpy as jnp
from jax.experimental import pallas as pl

_DIM = 384
_N = 1024
_K = 10
_PIX = 1024


def _block_body(x_ref, mem_ref, out_ref):
    x = x_ref[0]
    mem = mem_ref[...]
    l = jax.lax.dot_general(x, mem, (((0,), (1,)), ((), ())),
                            preferred_element_type=jnp.float32)
    half = _N // 2
    c1 = jnp.maximum(l[:, :half], l[:, half:])
    c2 = jnp.minimum(l[:, :half], l[:, half:])
    m = None
    for i in range(_K):
        t = jnp.max(c1, axis=1, keepdims=True)
        if i == 0:
            m = t
        hit = c1 >= t
        c1 = jnp.where(hit, c2, c1)
        c2 = jnp.where(hit, -jnp.inf, c2)
    e = jnp.exp(l - m)
    z = jnp.sum(e, axis=1, keepdims=True)
    p = e * (1.0 / z)
    w = jnp.where(l >= t, jnp.exp(p), 0.0)
    w = w * (1.0 / jnp.sum(w, axis=1, keepdims=True))
    out_ref[0] = jax.lax.dot_general(mem, w, (((0,), (1,)), ((), ())),
                                     preferred_element_type=jnp.float32)


def _branch(xf, mempool, interpret=False):
    b = xf.shape[0]
    return pl.pallas_call(
        _block_body,
        grid=(b,),
        in_specs=[
            pl.BlockSpec((1, _DIM, _PIX), lambda i: (i, 0, 0)),
            pl.BlockSpec((_N, _DIM), lambda i: (0, 0)),
        ],
        out_specs=pl.BlockSpec((1, _DIM, _PIX), lambda i: (i, 0, 0)),
        out_shape=jax.ShapeDtypeStruct((b, _DIM, _PIX), jnp.float32),
        interpret=interpret,
    )(xf, mempool)


def kernel(input1, input2, mempool):
    outs = []
    for x in (input1, input2):
        b, c, h, w = x.shape
        o = _branch(x.reshape(b, c, h * w), mempool)
        outs.append(o.reshape(b, c, h, w))
    return tuple(outs)


# speedup vs baseline: 1.6142x; 1.3831x over previous
import jax
import jax.numpy as jnp
from jax.experimental import pallas as pl

_DIM = 384
_N = 1024
_K = 10
_ROWS = 1024


def _block_body(q_ref, mem_ref, out_ref):
    q = q_ref[...]
    mem = mem_ref[...]
    l = jax.lax.dot_general(q, mem, (((1,), (1,)), ((), ())),
                            preferred_element_type=jnp.float32)
    half = _N // 2
    c1 = jnp.maximum(l[:, :half], l[:, half:])
    c2 = jnp.minimum(l[:, :half], l[:, half:])
    m = None
    for i in range(_K):
        t = jnp.max(c1, axis=1, keepdims=True)
        if i == 0:
            m = t
        hit = c1 >= t
        c1 = jnp.where(hit, c2, c1)
        c2 = jnp.where(hit, -jnp.inf, c2)
    e = jnp.exp(l - m)
    z = jnp.sum(e, axis=1, keepdims=True)
    et = jnp.exp(t - m)
    w = jnp.where(e >= et, jnp.exp(e * (1.0 / z)), 0.0)
    s = jnp.sum(w, axis=1, keepdims=True)
    o = jax.lax.dot_general(w, mem, (((1,), (0,)), ((), ())),
                            preferred_element_type=jnp.float32)
    out_ref[...] = o * (1.0 / s)


def _branch(q, mempool, interpret=False):
    rows = q.shape[0]
    return pl.pallas_call(
        _block_body,
        grid=(rows // _ROWS,),
        in_specs=[
            pl.BlockSpec((_ROWS, _DIM), lambda i: (i, 0)),
            pl.BlockSpec((_N, _DIM), lambda i: (0, 0)),
        ],
        out_specs=pl.BlockSpec((_ROWS, _DIM), lambda i: (i, 0)),
        out_shape=jax.ShapeDtypeStruct((rows, _DIM), jnp.float32),
        interpret=interpret,
    )(q, mempool)


def kernel(input1, input2, mempool):
    outs = []
    for x in (input1, input2):
        b, c, h, w = x.shape
        q = x.transpose(0, 2, 3, 1).reshape(-1, c)
        o = _branch(q, mempool)
        outs.append(o.reshape(b, h, w, c).transpose(0, 3, 1, 2))
    return tuple(outs)


# speedup vs baseline: 1.6168x; 1.0016x over previous
import jax
import jax.numpy as jnp
from jax.experimental import pallas as pl

_DIM = 384
_N = 1024
_K = 10
_ROWS = 1024


def _block_body(q_ref, mem_ref, out_ref):
    q = q_ref[...]
    mem = mem_ref[...]
    l = jax.lax.dot_general(q, mem, (((1,), (1,)), ((), ())),
                            preferred_element_type=jnp.float32)
    half = _N // 2
    c1 = jnp.maximum(l[:, :half], l[:, half:])
    c2 = jnp.minimum(l[:, :half], l[:, half:])
    m = None
    for i in range(_K):
        t = jnp.max(c1, axis=1, keepdims=True)
        if i == 0:
            m = t
        hit = c1 >= t
        c1 = jnp.where(hit, c2, c1)
        c2 = jnp.where(hit, -jnp.inf, c2)
    e = jnp.exp(l - m)
    z = jnp.sum(e, axis=1, keepdims=True)
    et = jnp.exp(t - m)
    w = jnp.where(e >= et, jnp.exp(e / z), 0.0)
    s = jnp.sum(w, axis=1, keepdims=True)
    o = jax.lax.dot_general(w, mem, (((1,), (0,)), ((), ())),
                            preferred_element_type=jnp.float32)
    out_ref[...] = o / s


def _branch(q, mempool, interpret=False):
    rows = q.shape[0]
    return pl.pallas_call(
        _block_body,
        grid=(rows // _ROWS,),
        in_specs=[
            pl.BlockSpec((_ROWS, _DIM), lambda i: (i, 0)),
            pl.BlockSpec((_N, _DIM), lambda i: (0, 0)),
        ],
        out_specs=pl.BlockSpec((_ROWS, _DIM), lambda i: (i, 0)),
        out_shape=jax.ShapeDtypeStruct((rows, _DIM), jnp.float32),
        interpret=interpret,
    )(q, mempool)


def kernel(input1, input2, mempool):
    outs = []
    for x in (input1, input2):
        b, c, h, w = x.shape
        q = x.transpose(0, 2, 3, 1).reshape(-1, c)
        o = _branch(q, mempool)
        outs.append(o.reshape(b, h, w, c).transpose(0, 3, 1, 2))
    return tuple(outs)
